# baseline (speedup 1.0000x reference)
import functools

import jax
import jax.numpy as jnp
from jax import lax
from jax.experimental import pallas as pl
from jax.experimental.pallas import tpu as pltpu
from jax.experimental.pallas import tpu_sc as plsc

_NUM_EMBEDDINGS = 100000
_DIM = 64
_BATCH = 1024


def _make_gather():
    info = plsc.get_sparse_core_info()
    nc, ns = info.num_cores, info.num_subcores
    nw = nc * ns
    b_per_w = _BATCH // nw
    mesh = plsc.VectorSubcoreMesh(core_axis_name="c", subcore_axis_name="s")

    @functools.partial(
        pl.kernel,
        mesh=mesh,
        out_type=jax.ShapeDtypeStruct((_BATCH, _DIM), jnp.float32),
        scratch_types=[
            pltpu.VMEM((b_per_w,), jnp.int32),
            pltpu.VMEM((b_per_w, _DIM), jnp.float32),
            pltpu.SemaphoreType.DMA,
        ],
        compiler_params=pltpu.CompilerParams(use_tc_tiling_on_sc=False),
    )
    def gather_kernel(idx_hbm, table_hbm, out_hbm, idx_v, rows_v, sem):
        wid = lax.axis_index("s") * nc + lax.axis_index("c")
        base = wid * b_per_w
        pltpu.sync_copy(idx_hbm.at[pl.ds(base, b_per_w)], idx_v)
        pltpu.async_copy(table_hbm.at[idx_v], rows_v, sem).wait()
        pltpu.sync_copy(rows_v, out_hbm.at[pl.ds(base, b_per_w)])

    return gather_kernel


_gather = _make_gather()


def kernel(token_ids, embedding):
    return _gather(token_ids.astype(jnp.int32), embedding)


# speedup vs baseline: 1.4836x; 1.4836x over previous
import functools

import jax
import jax.numpy as jnp
from jax import lax
from jax.experimental import pallas as pl
from jax.experimental.pallas import tpu as pltpu
from jax.experimental.pallas import tpu_sc as plsc

_NUM_EMBEDDINGS = 100000
_DIM = 64
_BATCH = 1024


def _make_gather():
    info = plsc.get_sparse_core_info()
    nc, ns = info.num_cores, info.num_subcores
    nw = nc * ns
    b_per_w = _BATCH // nw
    mesh = plsc.VectorSubcoreMesh(core_axis_name="c", subcore_axis_name="s")

    @functools.partial(
        pl.kernel,
        mesh=mesh,
        out_type=jax.ShapeDtypeStruct((_BATCH, _DIM), jnp.float32),
        scratch_types=[
            pltpu.VMEM((b_per_w,), jnp.int32),
            pltpu.VMEM((b_per_w, _DIM), jnp.float32),
            pltpu.SemaphoreType.DMA,
        ],
        compiler_params=pltpu.CompilerParams(needs_layout_passes=False),
    )
    def gather_kernel(idx_hbm, table_hbm, out_hbm, idx_v, rows_v, sem):
        wid = lax.axis_index("s") * nc + lax.axis_index("c")
        base = wid * b_per_w
        nl = 16
        pltpu.sync_copy(idx_hbm.at[pl.ds(base, b_per_w)], idx_v)
        lanes = lax.iota(jnp.int32, nl)
        copies = []
        for j in range(b_per_w):
            v = idx_v[pl.ds((j // nl) * nl, nl)]
            row = jnp.sum(jnp.where(lanes == (j % nl), v, 0), axis=0)
            copies.append(
                pltpu.async_copy(table_hbm.at[row], rows_v.at[j], sem))
        for c in copies:
            c.wait()
        pltpu.sync_copy(rows_v, out_hbm.at[pl.ds(base, b_per_w)])

    return gather_kernel


_gather = _make_gather()


def kernel(token_ids, embedding):
    return _gather(token_ids.astype(jnp.int32), embedding)


# speedup vs baseline: 1.4876x; 1.0028x over previous
import functools

import jax
import jax.numpy as jnp
from jax import lax
from jax.experimental import pallas as pl
from jax.experimental.pallas import tpu as pltpu
from jax.experimental.pallas import tpu_sc as plsc

_NUM_EMBEDDINGS = 100000
_DIM = 64
_BATCH = 1024


def _make_gather():
    info = plsc.get_sparse_core_info()
    nc, ns = info.num_cores, info.num_subcores
    nw = nc * ns
    b_per_w = _BATCH // nw
    mesh = plsc.VectorSubcoreMesh(core_axis_name="c", subcore_axis_name="s")

    @functools.partial(
        pl.kernel,
        mesh=mesh,
        out_type=jax.ShapeDtypeStruct((_BATCH, _DIM), jnp.float32),
        scratch_types=[
            pltpu.VMEM((b_per_w,), jnp.int32),
            pltpu.VMEM((b_per_w, _DIM), jnp.float32),
            pltpu.SemaphoreType.DMA,
        ],
    )
    def gather_kernel(idx_hbm, table_hbm, out_hbm, idx_v, rows_v, sem):
        wid = lax.axis_index("s") * nc + lax.axis_index("c")
        base = wid * b_per_w
        nl = 16
        pltpu.sync_copy(idx_hbm.at[pl.ds(base, b_per_w)], idx_v)
        copies = []
        for j in range(b_per_w):
            v = idx_v[pl.ds((j // nl) * nl, nl)]
            copies.append(
                pltpu.async_copy(table_hbm.at[v[j % nl]], rows_v.at[j], sem))
        for c in copies:
            c.wait()
        pltpu.sync_copy(rows_v, out_hbm.at[pl.ds(base, b_per_w)])

    return gather_kernel


_gather = _make_gather()


def kernel(token_ids, embedding):
    return _gather(token_ids.astype(jnp.int32), embedding)


# speedup vs baseline: 2.3290x; 1.5656x over previous
import functools

import jax
import jax.numpy as jnp
from jax import lax
from jax.experimental import pallas as pl
from jax.experimental.pallas import tpu as pltpu
from jax.experimental.pallas import tpu_sc as plsc

_NUM_EMBEDDINGS = 100000
_DIM = 64
_BATCH = 1024
_LANES = 128
_NB = 8


def _make_gather():
    info = plsc.get_sparse_core_info()
    nc, ns = info.num_cores, info.num_subcores
    nw = nc * ns
    b_per_w = _BATCH // nw
    nl = 16
    mesh = plsc.VectorSubcoreMesh(core_axis_name="c", subcore_axis_name="s")

    @functools.partial(
        pl.kernel,
        mesh=mesh,
        out_type=jax.ShapeDtypeStruct((_BATCH, _DIM), jnp.float32),
        scratch_types=[
            pltpu.VMEM((b_per_w,), jnp.int32),
            pltpu.VMEM((_NB, _DIM, _LANES), jnp.float32),
            pltpu.VMEM((b_per_w, _DIM), jnp.float32),
        ] + [pltpu.SemaphoreType.DMA] * _NB,
        compiler_params=pltpu.CompilerParams(needs_layout_passes=False),
    )
    def gather_kernel(idx_hbm, tablet_hbm, out_hbm,
                      idx_v, blk_v, rows_v, *sems):
        wid = lax.axis_index("s") * nc + lax.axis_index("c")
        base = wid * b_per_w
        pltpu.sync_copy(idx_hbm.at[pl.ds(base, b_per_w)], idx_v)
        lanes16 = lax.iota(jnp.int32, nl)
        handles = [None] * b_per_w

        def scalar_id(j):
            v = idx_v[pl.ds((j // nl) * nl, nl)]
            return v[j % nl]

        def fire(j):
            q = lax.shift_right_logical(scalar_id(j), 7)
            off = pl.multiple_of(q * _LANES, _LANES)
            handles[j] = pltpu.async_copy(
                tablet_hbm.at[:, pl.ds(off, _LANES)],
                blk_v.at[j % _NB], sems[j % _NB])

        for j in range(_NB):
            fire(j)
        for j in range(b_per_w):
            handles[j].wait()
            r = jnp.full((nl,), scalar_id(j) & (_LANES - 1), jnp.int32)
            blk_j = blk_v.at[j % _NB]
            for k in range(_DIM // nl):
                dd = lanes16 + (k * nl)
                val = plsc.load_gather(blk_j, [dd, r])
                rows_v[j, pl.ds(k * nl, nl)] = val
            if j + _NB < b_per_w:
                fire(j + _NB)
        pltpu.sync_copy(rows_v, out_hbm.at[pl.ds(base, b_per_w)])

    return gather_kernel


_gather = _make_gather()


def kernel(token_ids, embedding):
    return _gather(token_ids.astype(jnp.int32), embedding.T)


# speedup vs baseline: 2.3318x; 1.0012x over previous
import functools

import jax
import jax.numpy as jnp
from jax import lax
from jax.experimental import pallas as pl
from jax.experimental.pallas import tpu as pltpu
from jax.experimental.pallas import tpu_sc as plsc

_NUM_EMBEDDINGS = 100000
_DIM = 64
_BATCH = 1024
_LANES = 128
_NB = 12


def _make_gather():
    info = plsc.get_sparse_core_info()
    nc, ns = info.num_cores, info.num_subcores
    nw = nc * ns
    b_per_w = _BATCH // nw
    nl = 16
    mesh = plsc.VectorSubcoreMesh(core_axis_name="c", subcore_axis_name="s")

    @functools.partial(
        pl.kernel,
        mesh=mesh,
        out_type=jax.ShapeDtypeStruct((_BATCH, _DIM), jnp.float32),
        scratch_types=[
            pltpu.VMEM((b_per_w,), jnp.int32),
            pltpu.VMEM((_NB, _DIM, _LANES), jnp.float32),
            pltpu.VMEM((b_per_w, _DIM), jnp.float32),
        ] + [pltpu.SemaphoreType.DMA] * _NB,
        compiler_params=pltpu.CompilerParams(needs_layout_passes=False),
    )
    def gather_kernel(idx_hbm, tablet_hbm, out_hbm,
                      idx_v, blk_v, rows_v, *sems):
        wid = lax.axis_index("s") * nc + lax.axis_index("c")
        base = wid * b_per_w
        pltpu.sync_copy(idx_hbm.at[pl.ds(base, b_per_w)], idx_v)
        lanes16 = lax.iota(jnp.int32, nl)
        handles = [None] * b_per_w

        def scalar_id(j):
            v = idx_v[pl.ds((j // nl) * nl, nl)]
            return v[j % nl]

        def fire(j):
            q = lax.shift_right_logical(scalar_id(j), 7)
            off = pl.multiple_of(q * _LANES, _LANES)
            handles[j] = pltpu.async_copy(
                tablet_hbm.at[:, pl.ds(off, _LANES)],
                blk_v.at[j % _NB], sems[j % _NB])

        for j in range(_NB):
            fire(j)
        for j in range(b_per_w):
            handles[j].wait()
            r = jnp.full((nl,), scalar_id(j) & (_LANES - 1), jnp.int32)
            blk_j = blk_v.at[j % _NB]
            for k in range(_DIM // nl):
                dd = lanes16 + (k * nl)
                val = plsc.load_gather(blk_j, [dd, r])
                rows_v[j, pl.ds(k * nl, nl)] = val
            if j + _NB < b_per_w:
                fire(j + _NB)
        pltpu.sync_copy(rows_v, out_hbm.at[pl.ds(base, b_per_w)])

    return gather_kernel


_gather = _make_gather()


def kernel(token_ids, embedding):
    return _gather(token_ids.astype(jnp.int32), embedding.T)
